# Initial kernel scaffold; baseline (speedup 1.0000x reference)
import jax
import jax.numpy as jnp
from jax.experimental import pallas as pl


def kernel(map, room_mask, room_position_x, room_position_y, steps_remaining, door_connects, W_left, b_left, W_right, b_right, W_up, b_up, W_down, b_down, W_glob, b_glob, W_loc0, b_loc0, W_fc0, b_fc0, W_sv, b_sv):
    raise NotImplementedError("write your pallas kernel here")


# speedup vs baseline: 129.3956x; 129.3956x over previous
import jax
import jax.numpy as jnp
from jax.experimental import pallas as pl

_N = 64
_C = 4
_MX = 64
_MY = 64
_KS = 4
_L0 = 64
_G1 = 64
_NPIX = _MX * _MY


_SELU_ALPHA = 1.6732632423543772848170429916717
_SELU_SCALE = 1.0507009873554804934193349852946


def _selu(x):
    return _SELU_SCALE * jnp.where(
        x > 0, x, _SELU_ALPHA * (jnp.exp(x) - 1.0)
    )


def _shift_ax(a, axis, s):
    if s == 0:
        return a
    n = a.shape[axis]

    def sl(lo, hi):
        idx = [slice(None)] * a.ndim
        idx[axis] = slice(lo, hi)
        return a[tuple(idx)]

    if s < 0:
        edge = sl(0, 1)
        reps = [edge] * (-s)
        return jnp.concatenate(reps + [sl(0, n + s)], axis=axis)
    edge = sl(n - 1, n)
    reps = [edge] * s
    return jnp.concatenate([sl(s, n)] + reps, axis=axis)


def _prelude_kernel(gin_ref, wgT_ref, bg_ref, bT_ref, b2_ref, out_ref):
    gX = _selu(
        jnp.dot(gin_ref[...], wgT_ref[...], preferred_element_type=jnp.float32)
        + bg_ref[...]
    )
    out_ref[...] = (
        jnp.dot(gX, bT_ref[...], preferred_element_type=jnp.float32) + b2_ref[...]
    )


def _main_kernel(m3_ref, cenv_ref, wcatT_ref, bcat_ref, aT_ref, out_ref):
    m3 = m3_ref[0]
    ys = [_shift_ax(m3, 1, dy - 2) for dy in range(_KS)]
    y16 = jnp.concatenate(ys, axis=2)
    xs = [_shift_ax(y16, 0, dx - 2) for dx in range(_KS)]
    p3 = jnp.concatenate(xs, axis=2)
    P = p3.reshape(_NPIX, _KS * _KS * _C)

    lall = _selu(
        jnp.dot(P, wcatT_ref[...], preferred_element_type=jnp.float32)
        + bcat_ref[...]
    )

    f0 = (2 * _KS + 2) * _C
    ch1 = P[:, f0 + 1 : f0 + 2]
    ch2 = P[:, f0 + 2 : f0 + 3]
    masks = (ch1 > 1.0, ch1 < -1.0, ch2 > 1.0, ch2 < -1.0)

    cenv = cenv_ref[0]
    aT = aT_ref[...]
    total = jnp.zeros((1, _G1), jnp.float32)
    for d in range(4):
        ld = lall[:, d * _L0 : (d + 1) * _L0]
        gd = _selu(
            jnp.dot(ld, aT, preferred_element_type=jnp.float32) + cenv
        )
        total = total + jnp.sum(
            jnp.where(masks[d], gd, 0.0), axis=0, keepdims=True
        )
    out_ref[0] = total


def _tail_kernel(acc_ref, wfcT_ref, bfc_ref, wsvT_ref, bsv_ref, dc_ref, out_ref):
    g = _selu(
        jnp.dot(acc_ref[...], wfcT_ref[...], preferred_element_type=jnp.float32)
        + bfc_ref[...]
    )
    sv = jnp.dot(g, wsvT_ref[...], preferred_element_type=jnp.float32) + bsv_ref[...]
    door = jnp.where(dc_ref[...] > 0.5, 100000.0, sv[:, :512])
    allf = jnp.concatenate([door, sv[:, 512:]], axis=1)
    probs = jax.nn.sigmoid(allf)
    out_ref[...] = jnp.sum(probs, axis=1, keepdims=True) * 0.5


def kernel(map, room_mask, room_position_x, room_position_y, steps_remaining,
           door_connects, W_left, b_left, W_right, b_right, W_up, b_up,
           W_down, b_down, W_glob, b_glob, W_loc0, b_loc0, W_fc0, b_fc0,
           W_sv, b_sv):
    n = map.shape[0]
    f32 = jnp.float32

    m3 = map.astype(f32).transpose(0, 2, 3, 1)
    gin = jnp.concatenate(
        [room_mask.astype(f32), steps_remaining[:, None].astype(f32)], axis=1
    )
    wcatT = jnp.concatenate([W_left, W_right, W_up, W_down], axis=0).T
    bcat = jnp.concatenate([b_left, b_right, b_up, b_down])[None, :]
    aT = W_loc0[_G1:, :_L0].T
    bT = W_loc0[_G1:, _L0:].T
    b2 = b_loc0[_G1:][None, :]
    dc = door_connects.astype(f32)

    cenv = pl.pallas_call(
        _prelude_kernel,
        out_shape=jax.ShapeDtypeStruct((n, _G1), f32),
    )(gin, W_glob.T, b_glob[None, :], bT, b2)
    cenv3 = cenv.reshape(n, 1, _G1)

    acc3 = pl.pallas_call(
        _main_kernel,
        grid=(n,),
        in_specs=[
            pl.BlockSpec((1, _MX, _MY, _C), lambda e: (e, 0, 0, 0)),
            pl.BlockSpec((1, 1, _G1), lambda e: (e, 0, 0)),
            pl.BlockSpec((_L0, 4 * _L0), lambda e: (0, 0)),
            pl.BlockSpec((1, 4 * _L0), lambda e: (0, 0)),
            pl.BlockSpec((_L0, _G1), lambda e: (0, 0)),
        ],
        out_specs=pl.BlockSpec((1, 1, _G1), lambda e: (e, 0, 0)),
        out_shape=jax.ShapeDtypeStruct((n, 1, _G1), f32),
    )(m3, cenv3, wcatT, bcat, aT)
    acc = acc3.reshape(n, _G1)

    out = pl.pallas_call(
        _tail_kernel,
        out_shape=jax.ShapeDtypeStruct((n, 1), f32),
    )(acc, W_fc0.T, b_fc0[None, :], W_sv.T, b_sv[None, :], dc)
    return out[:, 0]


# speedup vs baseline: 162.7584x; 1.2578x over previous
import jax
import jax.numpy as jnp
from jax.experimental import pallas as pl

_N = 64
_C = 4
_MX = 64
_MY = 64
_KS = 4
_L0 = 64
_G1 = 64
_NPIX = _MX * _MY


_SELU_ALPHA = 1.6732632423543772848170429916717
_SELU_SCALE = 1.0507009873554804934193349852946


def _selu(x):
    return _SELU_SCALE * jnp.where(
        x > 0, x, _SELU_ALPHA * (jnp.exp(x) - 1.0)
    )


def _shift_ax(a, axis, s):
    if s == 0:
        return a
    n = a.shape[axis]

    def sl(lo, hi):
        idx = [slice(None)] * a.ndim
        idx[axis] = slice(lo, hi)
        return a[tuple(idx)]

    if s < 0:
        edge = sl(0, 1)
        reps = [edge] * (-s)
        return jnp.concatenate(reps + [sl(0, n + s)], axis=axis)
    edge = sl(n - 1, n)
    reps = [edge] * s
    return jnp.concatenate([sl(s, n)] + reps, axis=axis)


def _prelude_kernel(gin_ref, wgT_ref, bg_ref, bT_ref, b2_ref, out_ref):
    gX = _selu(
        jnp.dot(gin_ref[...], wgT_ref[...], preferred_element_type=jnp.float32)
        + bg_ref[...]
    )
    out_ref[...] = (
        jnp.dot(gX, bT_ref[...], preferred_element_type=jnp.float32) + b2_ref[...]
    )


def _main_kernel(m3_ref, cenvt_ref, wcatT_ref, bcat_ref, bd_ref, out_ref):
    m3 = m3_ref[0]
    ys = [_shift_ax(m3, 1, dy - 2) for dy in range(_KS)]
    y16 = jnp.concatenate(ys, axis=2)
    xs = [_shift_ax(y16, 0, dx - 2) for dx in range(_KS)]
    p3 = jnp.concatenate(xs, axis=2)
    P = p3.reshape(_NPIX, _KS * _KS * _C)

    lall = _selu(
        jnp.dot(P, wcatT_ref[...], preferred_element_type=jnp.float32)
        + bcat_ref[...]
    )

    f0 = (2 * _KS + 2) * _C
    ch1 = P[:, f0 + 1 : f0 + 2]
    ch2 = P[:, f0 + 2 : f0 + 3]
    masks = (ch1 > 1.0, ch1 < -1.0, ch2 > 1.0, ch2 < -1.0)

    gall = _selu(
        jnp.dot(lall, bd_ref[...], preferred_element_type=jnp.float32)
        + cenvt_ref[0]
    )
    maskw = jnp.concatenate(
        [jnp.broadcast_to(m.astype(jnp.float32), (_NPIX, _G1)) for m in masks],
        axis=1,
    )
    masked = gall * maskw
    ones_row = jnp.ones((1, _NPIX), jnp.float32)
    tot4 = jnp.dot(ones_row, masked, preferred_element_type=jnp.float32)
    out_ref[0] = (
        tot4[:, 0 * _G1 : 1 * _G1]
        + tot4[:, 1 * _G1 : 2 * _G1]
        + tot4[:, 2 * _G1 : 3 * _G1]
        + tot4[:, 3 * _G1 : 4 * _G1]
    )


def _tail_kernel(acc_ref, wfcT_ref, bfc_ref, wsvT_ref, bsv_ref, dc_ref, out_ref):
    g = _selu(
        jnp.dot(acc_ref[...], wfcT_ref[...], preferred_element_type=jnp.float32)
        + bfc_ref[...]
    )
    sv = jnp.dot(g, wsvT_ref[...], preferred_element_type=jnp.float32) + bsv_ref[...]
    door = jnp.where(dc_ref[...] > 0.5, 100000.0, sv[:, :512])
    allf = jnp.concatenate([door, sv[:, 512:]], axis=1)
    probs = jax.nn.sigmoid(allf)
    out_ref[...] = jnp.sum(probs, axis=1, keepdims=True) * 0.5


def kernel(map, room_mask, room_position_x, room_position_y, steps_remaining,
           door_connects, W_left, b_left, W_right, b_right, W_up, b_up,
           W_down, b_down, W_glob, b_glob, W_loc0, b_loc0, W_fc0, b_fc0,
           W_sv, b_sv):
    n = map.shape[0]
    f32 = jnp.float32

    m3 = map.astype(f32).transpose(0, 2, 3, 1)
    gin = jnp.concatenate(
        [room_mask.astype(f32), steps_remaining[:, None].astype(f32)], axis=1
    )
    wcatT = jnp.concatenate([W_left, W_right, W_up, W_down], axis=0).T
    bcat = jnp.concatenate([b_left, b_right, b_up, b_down])[None, :]
    aT = W_loc0[_G1:, :_L0].T
    bd = jax.scipy.linalg.block_diag(aT, aT, aT, aT)
    bT = W_loc0[_G1:, _L0:].T
    b2 = b_loc0[_G1:][None, :]
    dc = door_connects.astype(f32)

    cenv = pl.pallas_call(
        _prelude_kernel,
        out_shape=jax.ShapeDtypeStruct((n, _G1), f32),
    )(gin, W_glob.T, b_glob[None, :], bT, b2)
    cenvt3 = jnp.tile(cenv, (1, 4)).reshape(n, 1, 4 * _G1)

    acc3 = pl.pallas_call(
        _main_kernel,
        grid=(n,),
        in_specs=[
            pl.BlockSpec((1, _MX, _MY, _C), lambda e: (e, 0, 0, 0)),
            pl.BlockSpec((1, 1, 4 * _G1), lambda e: (e, 0, 0)),
            pl.BlockSpec((_L0, 4 * _L0), lambda e: (0, 0)),
            pl.BlockSpec((1, 4 * _L0), lambda e: (0, 0)),
            pl.BlockSpec((4 * _L0, 4 * _G1), lambda e: (0, 0)),
        ],
        out_specs=pl.BlockSpec((1, 1, _G1), lambda e: (e, 0, 0)),
        out_shape=jax.ShapeDtypeStruct((n, 1, _G1), f32),
    )(m3, cenvt3, wcatT, bcat, bd)
    acc = acc3.reshape(n, _G1)

    out = pl.pallas_call(
        _tail_kernel,
        out_shape=jax.ShapeDtypeStruct((n, 1), f32),
    )(acc, W_fc0.T, b_fc0[None, :], W_sv.T, b_sv[None, :], dc)
    return out[:, 0]


# speedup vs baseline: 339.2314x; 2.0843x over previous
import jax
import jax.numpy as jnp
from jax.experimental import pallas as pl

_N = 64
_C = 4
_MX = 64
_MY = 64
_KS = 4
_L0 = 64
_G1 = 64
_NPIX = _MX * _MY

_SELU_ALPHA = 1.6732632423543772848170429916717
_SELU_SCALE = 1.0507009873554804934193349852946


def _selu(x):
    one = jnp.asarray(1.0, x.dtype)
    return jnp.asarray(_SELU_SCALE, x.dtype) * jnp.where(
        x > 0, x, jnp.asarray(_SELU_ALPHA, x.dtype) * (jnp.exp(x) - one)
    )


def _lane_shift(a, s):
    n = a.shape[-1]
    if s == 0:
        return a
    if s > 0:
        return jnp.concatenate([jnp.broadcast_to(a[..., :1], a.shape[:-1] + (s,)), a[..., : n - s]], axis=-1)
    return jnp.concatenate([a[..., -s:], jnp.broadcast_to(a[..., -1:], a.shape[:-1] + (-s,))], axis=-1)


def _xshift(a, s):
    if s == 0:
        return a
    n = a.shape[-1]
    if s < 0:
        head = [a[..., :_MY]] * (-s)
        return jnp.concatenate(head + [a[..., : n + s * _MY]], axis=-1)
    tail = [a[..., n - _MY :]] * s
    return jnp.concatenate([a[..., s * _MY :]] + tail, axis=-1)


def _prelude_kernel(gin_ref, wgT_ref, bg_ref, bT_ref, b2_ref, out_ref):
    gX = _selu(
        jnp.dot(gin_ref[...], wgT_ref[...], preferred_element_type=jnp.float32)
        + bg_ref[...]
    )
    cenv = (
        jnp.dot(gX, bT_ref[...], preferred_element_type=jnp.float32) + b2_ref[...]
    )
    out_ref[...] = jnp.tile(cenv, (1, 4))


def _main_kernel(m4_ref, cenvT_ref, wcat_ref, bcat_ref, bdT_ref, out_ref):
    m4f = m4_ref[0]
    cenvT = cenvT_ref[0]

    mask_l = (m4f[1:2, :] > 1.0).astype(jnp.bfloat16)
    mask_r = (m4f[1:2, :] < -1.0).astype(jnp.bfloat16)
    mask_u = (m4f[2:3, :] > 1.0).astype(jnp.bfloat16)
    mask_d = (m4f[2:3, :] < -1.0).astype(jnp.bfloat16)

    m4 = m4f.astype(jnp.bfloat16)

    ymod = jax.lax.broadcasted_iota(jnp.int32, (_C, _NPIX), 1) % _MY
    sh_m1 = _lane_shift(m4, 1)
    sh_m2 = _lane_shift(m4, 2)
    sh_p1 = _lane_shift(m4, -1)
    y_sh = {
        -2: jnp.where(ymod >= 2, sh_m2, jnp.where(ymod == 1, sh_m1, m4)),
        -1: jnp.where(ymod >= 1, sh_m1, m4),
        0: m4,
        1: jnp.where(ymod <= _MY - 2, sh_p1, m4),
    }
    pieces = []
    for dx in range(_KS):
        for dy in range(_KS):
            pieces.append(_xshift(y_sh[dy - 2], dx - 2))
    PT = jnp.concatenate(pieces, axis=0)

    lallT = _selu(
        (
            jnp.dot(wcat_ref[...], PT, preferred_element_type=jnp.float32)
            + bcat_ref[...]
        ).astype(jnp.bfloat16)
    )

    gallT = _selu(
        (
            jnp.dot(bdT_ref[...], lallT, preferred_element_type=jnp.float32)
            + cenvT
        ).astype(jnp.bfloat16)
    )

    maskwT = jnp.concatenate(
        [
            jnp.broadcast_to(m, (_G1, _NPIX))
            for m in (mask_l, mask_r, mask_u, mask_d)
        ],
        axis=0,
    )
    maskedT = gallT * maskwT
    ones_col = jnp.ones((_NPIX, 1), jnp.bfloat16)
    tot4 = jnp.dot(maskedT, ones_col, preferred_element_type=jnp.float32)
    out_ref[0] = (
        tot4[0 * _G1 : 1 * _G1]
        + tot4[1 * _G1 : 2 * _G1]
        + tot4[2 * _G1 : 3 * _G1]
        + tot4[3 * _G1 : 4 * _G1]
    )


def _tail_kernel(acc_ref, wfcT_ref, bfc_ref, wsvT_ref, bsv_ref, dc_ref, out_ref):
    acc = acc_ref[...]
    g = _selu(
        jnp.dot(acc, wfcT_ref[...], preferred_element_type=jnp.float32)
        + bfc_ref[...]
    )
    sv = jnp.dot(g, wsvT_ref[...], preferred_element_type=jnp.float32) + bsv_ref[...]
    door = jnp.where(dc_ref[...] > 0.5, 100000.0, sv[:, :512])
    allf = jnp.concatenate([door, sv[:, 512:]], axis=1)
    probs = jax.nn.sigmoid(allf)
    out_ref[...] = jnp.sum(probs, axis=1, keepdims=True) * 0.5


def kernel(map, room_mask, room_position_x, room_position_y, steps_remaining,
           door_connects, W_left, b_left, W_right, b_right, W_up, b_up,
           W_down, b_down, W_glob, b_glob, W_loc0, b_loc0, W_fc0, b_fc0,
           W_sv, b_sv):
    n = map.shape[0]
    f32 = jnp.float32
    bf16 = jnp.bfloat16

    m4 = map.astype(f32).reshape(n, _C, _NPIX)
    gin = jnp.concatenate(
        [room_mask.astype(f32), steps_remaining[:, None].astype(f32)], axis=1
    )
    wcat = jnp.concatenate([W_left, W_right, W_up, W_down], axis=0).astype(bf16)
    bcat = jnp.concatenate([b_left, b_right, b_up, b_down])[:, None]
    aT = W_loc0[_G1:, :_L0]
    bdT = jax.scipy.linalg.block_diag(aT, aT, aT, aT).astype(bf16)
    bT = W_loc0[_G1:, _L0:].T
    b2 = b_loc0[_G1:][None, :]
    dc = door_connects.astype(f32)

    cenvt = pl.pallas_call(
        _prelude_kernel,
        out_shape=jax.ShapeDtypeStruct((n, 4 * _G1), f32),
    )(gin, W_glob.T, b_glob[None, :], bT, b2)
    cenvt3 = cenvt.reshape(n, 4 * _G1, 1)

    acc3 = pl.pallas_call(
        _main_kernel,
        grid=(n,),
        in_specs=[
            pl.BlockSpec((1, _C, _NPIX), lambda e: (e, 0, 0)),
            pl.BlockSpec((1, 4 * _G1, 1), lambda e: (e, 0, 0)),
            pl.BlockSpec((4 * _L0, _KS * _KS * _C), lambda e: (0, 0)),
            pl.BlockSpec((4 * _L0, 1), lambda e: (0, 0)),
            pl.BlockSpec((4 * _G1, 4 * _L0), lambda e: (0, 0)),
        ],
        out_specs=pl.BlockSpec((1, _G1, 1), lambda e: (e, 0, 0)),
        out_shape=jax.ShapeDtypeStruct((n, _G1, 1), f32),
    )(m4, cenvt3, wcat, bcat, bdT)

    out = pl.pallas_call(
        _tail_kernel,
        out_shape=jax.ShapeDtypeStruct((n, 1), f32),
    )(acc3.reshape(n, _G1), W_fc0.T, b_fc0[None, :], W_sv.T, b_sv[None, :], dc)
    return out[:, 0]


# speedup vs baseline: 340.5167x; 1.0038x over previous
import jax
import jax.numpy as jnp
from jax.experimental import pallas as pl
from jax.experimental.pallas import tpu as pltpu

_N = 64
_C = 4
_MX = 64
_MY = 64
_KS = 4
_L0 = 64
_G1 = 64
_NPIX = _MX * _MY

_SELU_ALPHA = 1.6732632423543772848170429916717
_SELU_SCALE = 1.0507009873554804934193349852946


def _selu(x):
    one = jnp.asarray(1.0, x.dtype)
    return jnp.asarray(_SELU_SCALE, x.dtype) * jnp.where(
        x > 0, x, jnp.asarray(_SELU_ALPHA, x.dtype) * (jnp.exp(x) - one)
    )


def _lane_shift(a, s):
    n = a.shape[-1]
    if s == 0:
        return a
    if s > 0:
        return jnp.concatenate([jnp.broadcast_to(a[..., :1], a.shape[:-1] + (s,)), a[..., : n - s]], axis=-1)
    return jnp.concatenate([a[..., -s:], jnp.broadcast_to(a[..., -1:], a.shape[:-1] + (-s,))], axis=-1)


def _xshift(a, s):
    if s == 0:
        return a
    n = a.shape[-1]
    if s < 0:
        head = [a[..., :_MY]] * (-s)
        return jnp.concatenate(head + [a[..., : n + s * _MY]], axis=-1)
    tail = [a[..., n - _MY :]] * s
    return jnp.concatenate([a[..., s * _MY :]] + tail, axis=-1)


def _prelude_kernel(gin_ref, wgT_ref, bg_ref, bT_ref, b2_ref, out_ref):
    gX = _selu(
        jnp.dot(gin_ref[...], wgT_ref[...], preferred_element_type=jnp.float32)
        + bg_ref[...]
    )
    cenv = (
        jnp.dot(gX, bT_ref[...], preferred_element_type=jnp.float32) + b2_ref[...]
    )
    out_ref[...] = jnp.tile(cenv, (1, 4))


def _main_kernel(m4_ref, cenvT_ref, wcat_ref, bcat_ref, bdT_ref, out_ref):
    m4f = m4_ref[0]
    cenvT = cenvT_ref[0]

    mask_l = (m4f[1:2, :] > 1.0).astype(jnp.bfloat16)
    mask_r = (m4f[1:2, :] < -1.0).astype(jnp.bfloat16)
    mask_u = (m4f[2:3, :] > 1.0).astype(jnp.bfloat16)
    mask_d = (m4f[2:3, :] < -1.0).astype(jnp.bfloat16)

    m4 = m4f.astype(jnp.bfloat16)

    ymod = jax.lax.broadcasted_iota(jnp.int32, (_C, _NPIX), 1) % _MY
    sh_m1 = _lane_shift(m4, 1)
    sh_m2 = _lane_shift(m4, 2)
    sh_p1 = _lane_shift(m4, -1)
    y_sh = {
        -2: jnp.where(ymod >= 2, sh_m2, jnp.where(ymod == 1, sh_m1, m4)),
        -1: jnp.where(ymod >= 1, sh_m1, m4),
        0: m4,
        1: jnp.where(ymod <= _MY - 2, sh_p1, m4),
    }
    pieces = []
    for dx in range(_KS):
        for dy in range(_KS):
            pieces.append(_xshift(y_sh[dy - 2], dx - 2))
    PT = jnp.concatenate(pieces, axis=0)

    lallT = _selu(
        (
            jnp.dot(wcat_ref[...], PT, preferred_element_type=jnp.float32)
            + bcat_ref[...]
        ).astype(jnp.bfloat16)
    )

    gallT = _selu(
        (
            jnp.dot(bdT_ref[...], lallT, preferred_element_type=jnp.float32)
            + cenvT
        ).astype(jnp.bfloat16)
    )

    maskwT = jnp.concatenate(
        [
            jnp.broadcast_to(m, (_G1, _NPIX))
            for m in (mask_l, mask_r, mask_u, mask_d)
        ],
        axis=0,
    )
    maskedT = gallT * maskwT
    ones_col = jnp.ones((_NPIX, 1), jnp.bfloat16)
    tot4 = jnp.dot(maskedT, ones_col, preferred_element_type=jnp.float32)
    out_ref[0] = (
        tot4[0 * _G1 : 1 * _G1]
        + tot4[1 * _G1 : 2 * _G1]
        + tot4[2 * _G1 : 3 * _G1]
        + tot4[3 * _G1 : 4 * _G1]
    )


def _tail_kernel(acc_ref, wfcT_ref, bfc_ref, wsvT_ref, bsv_ref, dc_ref, out_ref):
    acc = acc_ref[...]
    g = _selu(
        jnp.dot(acc, wfcT_ref[...], preferred_element_type=jnp.float32)
        + bfc_ref[...]
    )
    sv = jnp.dot(g, wsvT_ref[...], preferred_element_type=jnp.float32) + bsv_ref[...]
    door = jnp.where(dc_ref[...] > 0.5, 100000.0, sv[:, :512])
    allf = jnp.concatenate([door, sv[:, 512:]], axis=1)
    probs = jax.nn.sigmoid(allf)
    out_ref[...] = jnp.sum(probs, axis=1, keepdims=True) * 0.5


def kernel(map, room_mask, room_position_x, room_position_y, steps_remaining,
           door_connects, W_left, b_left, W_right, b_right, W_up, b_up,
           W_down, b_down, W_glob, b_glob, W_loc0, b_loc0, W_fc0, b_fc0,
           W_sv, b_sv):
    n = map.shape[0]
    f32 = jnp.float32
    bf16 = jnp.bfloat16

    m4 = map.astype(f32).reshape(n, _C, _NPIX)
    gin = jnp.concatenate(
        [room_mask.astype(f32), steps_remaining[:, None].astype(f32)], axis=1
    )
    wcat = jnp.concatenate([W_left, W_right, W_up, W_down], axis=0).astype(bf16)
    bcat = jnp.concatenate([b_left, b_right, b_up, b_down])[:, None]
    aT = W_loc0[_G1:, :_L0]
    bdT = jax.scipy.linalg.block_diag(aT, aT, aT, aT).astype(bf16)
    bT = W_loc0[_G1:, _L0:].T
    b2 = b_loc0[_G1:][None, :]
    dc = door_connects.astype(f32)

    cenvt = pl.pallas_call(
        _prelude_kernel,
        out_shape=jax.ShapeDtypeStruct((n, 4 * _G1), f32),
    )(gin, W_glob.T, b_glob[None, :], bT, b2)
    cenvt3 = cenvt.reshape(n, 4 * _G1, 1)

    acc3 = pl.pallas_call(
        _main_kernel,
        grid=(n,),
        in_specs=[
            pl.BlockSpec((1, _C, _NPIX), lambda e: (e, 0, 0)),
            pl.BlockSpec((1, 4 * _G1, 1), lambda e: (e, 0, 0)),
            pl.BlockSpec((4 * _L0, _KS * _KS * _C), lambda e: (0, 0)),
            pl.BlockSpec((4 * _L0, 1), lambda e: (0, 0)),
            pl.BlockSpec((4 * _G1, 4 * _L0), lambda e: (0, 0)),
        ],
        out_specs=pl.BlockSpec((1, _G1, 1), lambda e: (e, 0, 0)),
        out_shape=jax.ShapeDtypeStruct((n, _G1, 1), f32),
        compiler_params=pltpu.CompilerParams(
            dimension_semantics=("parallel",),
        ),
    )(m4, cenvt3, wcat, bcat, bdT)

    out = pl.pallas_call(
        _tail_kernel,
        out_shape=jax.ShapeDtypeStruct((n, 1), f32),
    )(acc3.reshape(n, _G1), W_fc0.T, b_fc0[None, :], W_sv.T, b_sv[None, :], dc)
    return out[:, 0]


# speedup vs baseline: 398.4652x; 1.1702x over previous
import jax
import jax.numpy as jnp
from jax.experimental import pallas as pl
from jax.experimental.pallas import tpu as pltpu

_N = 64
_C = 4
_MX = 64
_MY = 64
_KS = 4
_L0 = 64
_G1 = 64
_NPIX = _MX * _MY

_SELU_ALPHA = 1.6732632423543772848170429916717
_SELU_SCALE = 1.0507009873554804934193349852946


def _selu(x):
    one = jnp.asarray(1.0, x.dtype)
    return jnp.asarray(_SELU_SCALE, x.dtype) * jnp.where(
        x > 0, x, jnp.asarray(_SELU_ALPHA, x.dtype) * (jnp.exp(x) - one)
    )


def _lane_shift(a, s):
    n = a.shape[-1]
    if s == 0:
        return a
    if s > 0:
        return jnp.concatenate([jnp.broadcast_to(a[..., :1], a.shape[:-1] + (s,)), a[..., : n - s]], axis=-1)
    return jnp.concatenate([a[..., -s:], jnp.broadcast_to(a[..., -1:], a.shape[:-1] + (-s,))], axis=-1)


def _xshift(a, s):
    if s == 0:
        return a
    n = a.shape[-1]
    if s < 0:
        head = [a[..., :_MY]] * (-s)
        return jnp.concatenate(head + [a[..., : n + s * _MY]], axis=-1)
    tail = [a[..., n - _MY :]] * s
    return jnp.concatenate([a[..., s * _MY :]] + tail, axis=-1)


def _prelude_kernel(gin_ref, wgT_ref, bg_ref, bT_ref, b2_ref, out_ref):
    gX = _selu(
        jnp.dot(gin_ref[...], wgT_ref[...], preferred_element_type=jnp.float32)
        + bg_ref[...]
    )
    cenv = (
        jnp.dot(gX, bT_ref[...], preferred_element_type=jnp.float32) + b2_ref[...]
    )
    out_ref[...] = jnp.tile(cenv, (1, 4))


def _main_kernel(m4_ref, cenvT_ref, wcat_ref, bdT_ref, out_ref):
    m4f = m4_ref[0]
    cenvT = cenvT_ref[0]

    mask_l = (m4f[1:2, :] > 1.0).astype(jnp.bfloat16)
    mask_r = (m4f[1:2, :] < -1.0).astype(jnp.bfloat16)
    mask_u = (m4f[2:3, :] > 1.0).astype(jnp.bfloat16)
    mask_d = (m4f[2:3, :] < -1.0).astype(jnp.bfloat16)

    m4 = m4f.astype(jnp.bfloat16)

    ymod = jax.lax.broadcasted_iota(jnp.int32, (_C, _NPIX), 1) % _MY
    sh_m1 = _lane_shift(m4, 1)
    sh_m2 = _lane_shift(m4, 2)
    sh_p1 = _lane_shift(m4, -1)
    y_sh = {
        -2: jnp.where(ymod >= 2, sh_m2, jnp.where(ymod == 1, sh_m1, m4)),
        -1: jnp.where(ymod >= 1, sh_m1, m4),
        0: m4,
        1: jnp.where(ymod <= _MY - 2, sh_p1, m4),
    }
    pieces = []
    for dx in range(_KS):
        for dy in range(_KS):
            pieces.append(_xshift(y_sh[dy - 2], dx - 2))
    PT = jnp.concatenate(pieces, axis=0)

    PTb = jnp.concatenate(
        [PT, jnp.ones((1, _NPIX), jnp.bfloat16)], axis=0
    )
    lallT = _selu(
        jnp.dot(
            wcat_ref[...], PTb, preferred_element_type=jnp.float32
        ).astype(jnp.bfloat16)
    )

    gallT = _selu(
        (
            jnp.dot(bdT_ref[...], lallT, preferred_element_type=jnp.float32)
            + cenvT
        ).astype(jnp.bfloat16)
    )

    maskwT = jnp.concatenate(
        [
            jnp.broadcast_to(m, (_G1, _NPIX))
            for m in (mask_l, mask_r, mask_u, mask_d)
        ],
        axis=0,
    )
    maskedT = gallT * maskwT
    ones_col = jnp.ones((_NPIX, 1), jnp.bfloat16)
    tot4 = jnp.dot(maskedT, ones_col, preferred_element_type=jnp.float32)
    out_ref[0] = (
        tot4[0 * _G1 : 1 * _G1]
        + tot4[1 * _G1 : 2 * _G1]
        + tot4[2 * _G1 : 3 * _G1]
        + tot4[3 * _G1 : 4 * _G1]
    )


def _tail_kernel(acc_ref, wfcT_ref, bfc_ref, wsvT_ref, bsv_ref, dc_ref, out_ref):
    acc = acc_ref[...]
    g = _selu(
        jnp.dot(acc, wfcT_ref[...], preferred_element_type=jnp.float32)
        + bfc_ref[...]
    )
    sv = jnp.dot(g, wsvT_ref[...], preferred_element_type=jnp.float32) + bsv_ref[...]
    door = jnp.where(dc_ref[...] > 0.5, 100000.0, sv[:, :512])
    allf = jnp.concatenate([door, sv[:, 512:]], axis=1)
    probs = jax.nn.sigmoid(allf)
    out_ref[...] = jnp.sum(probs, axis=1, keepdims=True) * 0.5


def kernel(map, room_mask, room_position_x, room_position_y, steps_remaining,
           door_connects, W_left, b_left, W_right, b_right, W_up, b_up,
           W_down, b_down, W_glob, b_glob, W_loc0, b_loc0, W_fc0, b_fc0,
           W_sv, b_sv):
    n = map.shape[0]
    f32 = jnp.float32
    bf16 = jnp.bfloat16

    m4 = map.astype(f32).reshape(n, _C, _NPIX)
    gin = jnp.concatenate(
        [room_mask.astype(f32), steps_remaining[:, None].astype(f32)], axis=1
    )
    wcat = jnp.concatenate(
        [
            jnp.concatenate([W_left, W_right, W_up, W_down], axis=0),
            jnp.concatenate([b_left, b_right, b_up, b_down])[:, None],
        ],
        axis=1,
    ).astype(bf16)
    aT = W_loc0[_G1:, :_L0]
    bdT = jax.scipy.linalg.block_diag(aT, aT, aT, aT).astype(bf16)
    bT = W_loc0[_G1:, _L0:].T
    b2 = b_loc0[_G1:][None, :]
    dc = door_connects.astype(f32)

    cenvt = pl.pallas_call(
        _prelude_kernel,
        out_shape=jax.ShapeDtypeStruct((n, 4 * _G1), f32),
    )(gin, W_glob.T, b_glob[None, :], bT, b2)
    cenvt3 = cenvt.reshape(n, 4 * _G1, 1).astype(bf16)

    acc3 = pl.pallas_call(
        _main_kernel,
        grid=(n,),
        in_specs=[
            pl.BlockSpec((1, _C, _NPIX), lambda e: (e, 0, 0)),
            pl.BlockSpec((1, 4 * _G1, 1), lambda e: (e, 0, 0)),
            pl.BlockSpec((4 * _L0, _KS * _KS * _C + 1), lambda e: (0, 0)),
            pl.BlockSpec((4 * _G1, 4 * _L0), lambda e: (0, 0)),
        ],
        out_specs=pl.BlockSpec((1, _G1, 1), lambda e: (e, 0, 0)),
        out_shape=jax.ShapeDtypeStruct((n, _G1, 1), f32),
        compiler_params=pltpu.CompilerParams(
            dimension_semantics=("parallel",),
        ),
    )(m4, cenvt3, wcat, bdT)

    out = pl.pallas_call(
        _tail_kernel,
        out_shape=jax.ShapeDtypeStruct((n, 1), f32),
    )(acc3.reshape(n, _G1), W_fc0.T, b_fc0[None, :], W_sv.T, b_sv[None, :], dc)
    return out[:, 0]


# speedup vs baseline: 491.4111x; 1.2333x over previous
import jax
import jax.numpy as jnp
from jax.experimental import pallas as pl
from jax.experimental.pallas import tpu as pltpu

_N = 64
_C = 4
_MX = 64
_MY = 64
_KS = 4
_L0 = 64
_G1 = 64
_NPIX = _MX * _MY
_EPB = 8

_SELU_ALPHA = 1.6732632423543772848170429916717
_SELU_SCALE = 1.0507009873554804934193349852946


def _selu(x):
    sa = jnp.asarray(_SELU_SCALE * _SELU_ALPHA, x.dtype)
    sc = jnp.asarray(_SELU_SCALE, x.dtype)
    return jnp.where(x > 0, sc * x, sa * jnp.exp(x) - sa)


def _selu_noscale(x):
    a = jnp.asarray(_SELU_ALPHA, x.dtype)
    return jnp.where(x > 0, x, a * jnp.exp(x) - a)


def _lane_shift(a, s):
    n = a.shape[-1]
    if s == 0:
        return a
    if s > 0:
        return jnp.concatenate([jnp.broadcast_to(a[..., :1], a.shape[:-1] + (s,)), a[..., : n - s]], axis=-1)
    return jnp.concatenate([a[..., -s:], jnp.broadcast_to(a[..., -1:], a.shape[:-1] + (-s,))], axis=-1)


def _xshift(a, s):
    if s == 0:
        return a
    n = a.shape[-1]
    if s < 0:
        head = [a[..., :_MY]] * (-s)
        return jnp.concatenate(head + [a[..., : n + s * _MY]], axis=-1)
    tail = [a[..., n - _MY :]] * s
    return jnp.concatenate([a[..., s * _MY :]] + tail, axis=-1)


def _fused_kernel(
    m4_ref, ginT_ref, wg_ref, bgc_ref, bmat_ref, b2c_ref,
    wcat_ref, bdT_ref, wfcT_ref, bfc_ref, wsvT_ref, bsv_ref, dc_ref,
    out_ref, cenv_scr, acc_scr,
):
    pid = pl.program_id(0)
    nprog = pl.num_programs(0)

    @pl.when(pid == 0)
    def _prelude():
        gXT = _selu(
            jnp.dot(wg_ref[...], ginT_ref[...], preferred_element_type=jnp.float32)
            + bgc_ref[...]
        )
        cT = (
            jnp.dot(bmat_ref[...], gXT, preferred_element_type=jnp.float32)
            + b2c_ref[...]
        )
        cenv_scr[...] = jnp.concatenate([cT, cT], axis=0).astype(jnp.bfloat16)

    lane8 = jax.lax.broadcasted_iota(jnp.int32, (_N, _EPB), 1)
    sub64 = jax.lax.broadcasted_iota(jnp.int32, (_N, _EPB), 0)
    onehot = (sub64 == pid * _EPB + lane8).astype(jnp.bfloat16)
    cenv8 = jnp.dot(
        cenv_scr[...], onehot, preferred_element_type=jnp.float32
    ).astype(jnp.bfloat16)

    tots = []
    for ei in range(_EPB):
        tots.append(
            _one_env(m4_ref[ei], cenv8[:, ei : ei + 1], wcat_ref, bdT_ref)
        )
    tot8 = jnp.concatenate(tots, axis=1)

    eye = (
        jax.lax.broadcasted_iota(jnp.int32, (_G1, _G1), 0)
        == jax.lax.broadcasted_iota(jnp.int32, (_G1, _G1), 1)
    ).astype(jnp.bfloat16)
    tot8row = jax.lax.dot_general(
        tot8.astype(jnp.bfloat16), eye,
        (((0,), (0,)), ((), ())),
        preferred_element_type=jnp.float32,
    )
    acc_scr[pl.ds(pid * _EPB, _EPB), :] = _SELU_SCALE * tot8row

    @pl.when(pid == nprog - 1)
    def _tail():
        g = _selu(
            jnp.dot(acc_scr[...], wfcT_ref[...], preferred_element_type=jnp.float32)
            + bfc_ref[...]
        )
        sv = (
            jnp.dot(g, wsvT_ref[...], preferred_element_type=jnp.float32)
            + bsv_ref[...]
        )
        door = jnp.where(dc_ref[...] > 0.5, 100000.0, sv[:, :512])
        allf = jnp.concatenate([door, sv[:, 512:]], axis=1)
        probs = jax.nn.sigmoid(allf)
        out_ref[...] = jnp.sum(probs, axis=1, keepdims=True) * 0.5


def _one_env(m4f, cenvT, wcat_ref, bdT_ref):

    mask_l = (m4f[1:2, :] > 1.0).astype(jnp.bfloat16)
    mask_r = (m4f[1:2, :] < -1.0).astype(jnp.bfloat16)
    mask_u = (m4f[2:3, :] > 1.0).astype(jnp.bfloat16)
    mask_d = (m4f[2:3, :] < -1.0).astype(jnp.bfloat16)

    m4 = m4f.astype(jnp.bfloat16)

    ymod = jax.lax.broadcasted_iota(jnp.int32, (_C, _NPIX), 1) % _MY
    sh_m1 = _lane_shift(m4, 1)
    sh_m2 = _lane_shift(m4, 2)
    sh_p1 = _lane_shift(m4, -1)
    y_sh = {
        -2: jnp.where(ymod >= 2, sh_m2, jnp.where(ymod == 1, sh_m1, m4)),
        -1: jnp.where(ymod >= 1, sh_m1, m4),
        0: m4,
        1: jnp.where(ymod <= _MY - 2, sh_p1, m4),
    }
    y01 = jnp.concatenate([y_sh[-2], y_sh[-1]], axis=0)
    y23 = jnp.concatenate([y_sh[0], y_sh[1]], axis=0)
    pieces = []
    for dx in range(_KS):
        pieces.append(_xshift(y01, dx - 2))
        pieces.append(_xshift(y23, dx - 2))
    PT = jnp.concatenate(pieces, axis=0)

    PTb = jnp.concatenate(
        [PT, jnp.ones((1, _NPIX), jnp.bfloat16)], axis=0
    )
    lallT = _selu_noscale(
        jnp.dot(
            wcat_ref[...], PTb, preferred_element_type=jnp.float32
        ).astype(jnp.bfloat16)
    )

    bd2 = bdT_ref[...]
    g_top = _selu_noscale(
        jnp.dot(bd2, lallT[: 2 * _L0], preferred_element_type=jnp.float32)
        .astype(jnp.bfloat16) + cenvT
    )
    g_bot = _selu_noscale(
        jnp.dot(bd2, lallT[2 * _L0 :], preferred_element_type=jnp.float32)
        .astype(jnp.bfloat16) + cenvT
    )

    fold = (
        mask_l * g_top[:_G1]
        + mask_r * g_top[_G1:]
        + mask_u * g_bot[:_G1]
        + mask_d * g_bot[_G1:]
    )
    ones_col = jnp.ones((_NPIX, 1), jnp.bfloat16)
    return jnp.dot(fold, ones_col, preferred_element_type=jnp.float32)


def kernel(map, room_mask, room_position_x, room_position_y, steps_remaining,
           door_connects, W_left, b_left, W_right, b_right, W_up, b_up,
           W_down, b_down, W_glob, b_glob, W_loc0, b_loc0, W_fc0, b_fc0,
           W_sv, b_sv):
    n = map.shape[0]
    f32 = jnp.float32
    bf16 = jnp.bfloat16

    m4 = map.astype(f32).reshape(n, _C, _NPIX)
    ginT = jnp.concatenate(
        [room_mask.astype(f32), steps_remaining[:, None].astype(f32)], axis=1
    ).T
    wcat = jnp.concatenate(
        [
            jnp.concatenate([W_left, W_right, W_up, W_down], axis=0),
            jnp.concatenate([b_left, b_right, b_up, b_down])[:, None],
        ],
        axis=1,
    ).astype(bf16)
    aT = W_loc0[_G1:, :_L0]
    bdT = (_SELU_SCALE * jax.scipy.linalg.block_diag(aT, aT)).astype(bf16)
    bmat = W_loc0[_G1:, _L0:]
    b2c = b_loc0[_G1:][:, None]
    dc = door_connects.astype(f32)

    full = lambda shape: pl.BlockSpec(shape, lambda e: tuple(0 for _ in shape))
    out = pl.pallas_call(
        _fused_kernel,
        grid=(n // _EPB,),
        in_specs=[
            pl.BlockSpec((_EPB, _C, _NPIX), lambda e: (e, 0, 0)),
            full(ginT.shape),
            full(W_glob.shape),
            full((_G1, 1)),
            full(bmat.shape),
            full((_G1, 1)),
            full(wcat.shape),
            full(bdT.shape),
            full((_G1, 128)),
            full((1, 128)),
            full((128, 640)),
            full((1, 640)),
            full(dc.shape),
        ],
        out_specs=pl.BlockSpec((n, 1), lambda e: (0, 0)),
        out_shape=jax.ShapeDtypeStruct((n, 1), f32),
        scratch_shapes=[
            pltpu.VMEM((2 * _G1, n), bf16),
            pltpu.VMEM((n, _G1), f32),
        ],
    )(
        m4, ginT, W_glob, b_glob[:, None], bmat, b2c,
        wcat, bdT, W_fc0.T, b_fc0[None, :], W_sv.T, b_sv[None, :], dc,
    )
    return out[:, 0]


# speedup vs baseline: 494.9811x; 1.0073x over previous
import jax
import jax.numpy as jnp
from jax.experimental import pallas as pl
from jax.experimental.pallas import tpu as pltpu

_N = 64
_C = 4
_MX = 64
_MY = 64
_KS = 4
_L0 = 64
_G1 = 64
_NPIX = _MX * _MY
_EPB = 8

_SELU_ALPHA = 1.6732632423543772848170429916717
_SELU_SCALE = 1.0507009873554804934193349852946


def _selu(x):
    sa = jnp.asarray(_SELU_SCALE * _SELU_ALPHA, x.dtype)
    sc = jnp.asarray(_SELU_SCALE, x.dtype)
    return jnp.where(x > 0, sc * x, sa * jnp.exp(x) - sa)


def _selu_noscale(x):
    a = jnp.asarray(_SELU_ALPHA, x.dtype)
    return jnp.where(x > 0, x, a * jnp.exp(x) - a)


def _lane_shift(a, s):
    n = a.shape[-1]
    if s == 0:
        return a
    if s > 0:
        return jnp.concatenate([jnp.broadcast_to(a[..., :1], a.shape[:-1] + (s,)), a[..., : n - s]], axis=-1)
    return jnp.concatenate([a[..., -s:], jnp.broadcast_to(a[..., -1:], a.shape[:-1] + (-s,))], axis=-1)


def _xshift(a, s):
    if s == 0:
        return a
    n = a.shape[-1]
    if s < 0:
        head = [a[..., :_MY]] * (-s)
        return jnp.concatenate(head + [a[..., : n + s * _MY]], axis=-1)
    tail = [a[..., n - _MY :]] * s
    return jnp.concatenate([a[..., s * _MY :]] + tail, axis=-1)


def _prelude_kernel(gin_ref, wgT_ref, bg_ref, bT_ref, b2_ref, out_ref):
    gX = _selu(
        jnp.dot(gin_ref[...], wgT_ref[...], preferred_element_type=jnp.float32)
        + bg_ref[...]
    )
    cenv = (
        jnp.dot(gX, bT_ref[...], preferred_element_type=jnp.float32) + b2_ref[...]
    )
    out_ref[...] = jnp.tile(cenv, (1, 2))


def _main_kernel(m4_ref, cenvT_ref, wcat_ref, bdT_ref, out_ref):
    for ei in range(_EPB):
        _one_env(
            m4_ref[ei], cenvT_ref[ei], wcat_ref, bdT_ref, out_ref, ei
        )


def _one_env(m4f, cenvT, wcat_ref, bdT_ref, out_ref, ei):

    mask_l = (m4f[1:2, :] > 1.0).astype(jnp.bfloat16)
    mask_r = (m4f[1:2, :] < -1.0).astype(jnp.bfloat16)
    mask_u = (m4f[2:3, :] > 1.0).astype(jnp.bfloat16)
    mask_d = (m4f[2:3, :] < -1.0).astype(jnp.bfloat16)

    m4 = m4f.astype(jnp.bfloat16)

    ymod = jax.lax.broadcasted_iota(jnp.int32, (_C, _NPIX), 1) % _MY
    sh_m1 = _lane_shift(m4, 1)
    sh_m2 = _lane_shift(m4, 2)
    sh_p1 = _lane_shift(m4, -1)
    y_sh = {
        -2: jnp.where(ymod >= 2, sh_m2, jnp.where(ymod == 1, sh_m1, m4)),
        -1: jnp.where(ymod >= 1, sh_m1, m4),
        0: m4,
        1: jnp.where(ymod <= _MY - 2, sh_p1, m4),
    }
    y01 = jnp.concatenate([y_sh[-2], y_sh[-1]], axis=0)
    y23 = jnp.concatenate([y_sh[0], y_sh[1]], axis=0)
    pieces = []
    for dx in range(_KS):
        pieces.append(_xshift(y01, dx - 2))
        pieces.append(_xshift(y23, dx - 2))
    PT = jnp.concatenate(pieces, axis=0)

    PTb = jnp.concatenate(
        [PT, jnp.ones((1, _NPIX), jnp.bfloat16)], axis=0
    )
    lallT = _selu_noscale(
        jnp.dot(
            wcat_ref[...], PTb, preferred_element_type=jnp.float32
        ).astype(jnp.bfloat16)
    )

    bd2 = bdT_ref[...]
    g_top = _selu_noscale(
        jnp.dot(bd2, lallT[: 2 * _L0], preferred_element_type=jnp.float32)
        .astype(jnp.bfloat16) + cenvT
    )
    g_bot = _selu_noscale(
        jnp.dot(bd2, lallT[2 * _L0 :], preferred_element_type=jnp.float32)
        .astype(jnp.bfloat16) + cenvT
    )

    fold = (
        mask_l * g_top[:_G1]
        + mask_r * g_top[_G1:]
        + mask_u * g_bot[:_G1]
        + mask_d * g_bot[_G1:]
    )
    ones_col = jnp.ones((_NPIX, 1), jnp.bfloat16)
    out_ref[ei] = _SELU_SCALE * jnp.dot(
        fold, ones_col, preferred_element_type=jnp.float32
    )


def _tail_kernel(acc_ref, wfcT_ref, bfc_ref, wsvT_ref, bsv_ref, dc_ref, out_ref):
    acc = acc_ref[...]
    g = _selu(
        jnp.dot(acc, wfcT_ref[...], preferred_element_type=jnp.float32)
        + bfc_ref[...]
    )
    sv = jnp.dot(g, wsvT_ref[...], preferred_element_type=jnp.float32) + bsv_ref[...]
    door = jnp.where(dc_ref[...] > 0.5, 100000.0, sv[:, :512])
    allf = jnp.concatenate([door, sv[:, 512:]], axis=1)
    probs = jax.nn.sigmoid(allf)
    out_ref[...] = jnp.sum(probs, axis=1, keepdims=True) * 0.5


def kernel(map, room_mask, room_position_x, room_position_y, steps_remaining,
           door_connects, W_left, b_left, W_right, b_right, W_up, b_up,
           W_down, b_down, W_glob, b_glob, W_loc0, b_loc0, W_fc0, b_fc0,
           W_sv, b_sv):
    n = map.shape[0]
    f32 = jnp.float32
    bf16 = jnp.bfloat16

    m4 = map.astype(f32).reshape(n, _C, _NPIX)
    gin = jnp.concatenate(
        [room_mask.astype(f32), steps_remaining[:, None].astype(f32)], axis=1
    )
    wcat = jnp.concatenate(
        [
            jnp.concatenate([W_left, W_right, W_up, W_down], axis=0),
            jnp.concatenate([b_left, b_right, b_up, b_down])[:, None],
        ],
        axis=1,
    ).astype(bf16)
    aT = W_loc0[_G1:, :_L0]
    bdT = (_SELU_SCALE * jax.scipy.linalg.block_diag(aT, aT)).astype(bf16)
    bT = W_loc0[_G1:, _L0:].T
    b2 = b_loc0[_G1:][None, :]
    dc = door_connects.astype(f32)

    cenvt = pl.pallas_call(
        _prelude_kernel,
        out_shape=jax.ShapeDtypeStruct((n, 2 * _G1), f32),
    )(gin, W_glob.T, b_glob[None, :], bT, b2)
    cenvt3 = cenvt.reshape(n, 2 * _G1, 1).astype(bf16)

    acc3 = pl.pallas_call(
        _main_kernel,
        grid=(n // _EPB,),
        in_specs=[
            pl.BlockSpec((_EPB, _C, _NPIX), lambda e: (e, 0, 0)),
            pl.BlockSpec((_EPB, 2 * _G1, 1), lambda e: (e, 0, 0)),
            pl.BlockSpec((4 * _L0, _KS * _KS * _C + 1), lambda e: (0, 0)),
            pl.BlockSpec((2 * _G1, 2 * _L0), lambda e: (0, 0)),
        ],
        out_specs=pl.BlockSpec((_EPB, _G1, 1), lambda e: (e, 0, 0)),
        out_shape=jax.ShapeDtypeStruct((n, _G1, 1), f32),
        compiler_params=pltpu.CompilerParams(
            dimension_semantics=("parallel",),
        ),
    )(m4, cenvt3, wcat, bdT)

    out = pl.pallas_call(
        _tail_kernel,
        out_shape=jax.ShapeDtypeStruct((n, 1), f32),
    )(acc3.reshape(n, _G1), W_fc0.T, b_fc0[None, :], W_sv.T, b_sv[None, :], dc)
    return out[:, 0]
